# Initial kernel scaffold; baseline (speedup 1.0000x reference)
import jax
import jax.numpy as jnp
from jax.experimental import pallas as pl


def kernel(edge_index, radial_feature, node_feats):
    raise NotImplementedError("write your pallas kernel here")


# speedup vs baseline: 33.3726x; 33.3726x over previous
import jax
import jax.numpy as jnp
from jax import lax
from jax.experimental import pallas as pl
from jax.experimental.pallas import tpu as pltpu
from jax.experimental.pallas import tpu_sc as plsc

N_NODES = 50000
N_EDGES = 800000
PAYLOAD = 32

NUM_SUBCORES = 16
EDGES_PER_TILE = N_EDGES // NUM_SUBCORES
CHUNK = 400
CHUNKS_PER_TILE = EDGES_PER_TILE // CHUNK
GROUPS = CHUNK // 16
SCATTER_ROWS = 100
SCATTERS = CHUNK // SCATTER_ROWS
N_PAD = 51200
ROWS_PER_TILE = N_PAD // NUM_SUBCORES


def _sums_body(x_ref, o_ref):
    o_ref[...] = jnp.sum(x_ref[...], axis=1, keepdims=True)


def _node_sums(node_feats):
    n = node_feats.shape[0]
    out = pl.pallas_call(
        _sums_body,
        grid=(50,),
        in_specs=[pl.BlockSpec((n // 50, 128), lambda i: (i, 0))],
        out_specs=pl.BlockSpec((n // 50, 1), lambda i: (i, 0)),
        out_shape=jax.ShapeDtypeStruct((n, 1), jnp.float32),
    )(node_feats)
    return out.reshape(n)


def _sc_body(sums_hbm, snd_hbm, rcv_hbm, radial_hbm, a_hbm, comb_hbm,
             buf, snd_v, rcv_v, s_v, sums_sh, a_sh):
    c = lax.axis_index("c")
    s = lax.axis_index("s")
    zeros16 = jnp.zeros((16,), jnp.float32)
    lanes = lax.iota(jnp.int32, 16)
    e_base = s * EDGES_PER_TILE
    rz = s * ROWS_PER_TILE

    def zrow(i, carry):
        buf[i, pl.ds(0, 16)] = zeros16
        buf[i, pl.ds(16, 16)] = zeros16
        return carry

    lax.fori_loop(0, CHUNK, zrow, 0)
    for i in range(ROWS_PER_TILE // CHUNK):
        pltpu.sync_copy(buf.at[pl.ds(0, CHUNK)],
                        a_sh.at[pl.ds(rz + i * CHUNK, CHUNK)])

    @pl.when(s == 0)
    def _():
        pltpu.sync_copy(sums_hbm, sums_sh)

    plsc.subcore_barrier()

    def chunk_body(k, carry):
        e0 = e_base + k * CHUNK
        cid = e0 // CHUNK
        pltpu.sync_copy(rcv_hbm.at[cid], rcv_v)
        pltpu.sync_copy(snd_hbm.at[pl.ds(e0, CHUNK)], snd_v)
        pltpu.sync_copy(radial_hbm.at[c, pl.ds(e0, CHUNK)], buf)
        pltpu.sync_copy(sums_sh.at[snd_v], s_v)

        def grp(g, gc):
            b = g * 16
            s16 = s_v[pl.ds(b, 16)]
            rows = lanes + b
            for j in range(PAYLOAD):
                cols = jnp.full((16,), j, jnp.int32)
                v = plsc.load_gather(buf, [rows, cols])
                plsc.store_scatter(buf, [rows, cols], v * s16)
            return gc

        lax.fori_loop(0, GROUPS, grp, 0)

        pltpu.sync_copy(buf, comb_hbm.at[c, pl.ds(e0, CHUNK)])
        for j in range(SCATTERS):
            pltpu.sync_copy(buf.at[pl.ds(j * SCATTER_ROWS, SCATTER_ROWS)],
                            a_sh.at[rcv_v.at[j]], add=True)
        return carry

    lax.fori_loop(0, CHUNKS_PER_TILE, chunk_body, 0)
    plsc.subcore_barrier()
    pltpu.sync_copy(a_sh.at[pl.ds(rz, ROWS_PER_TILE)],
                    a_hbm.at[c, pl.ds(rz, ROWS_PER_TILE)])


def _sc_main(sums, sender, rcv3d, radial):
    run = pl.kernel(
        _sc_body,
        out_type=[
            jax.ShapeDtypeStruct((2, N_PAD, PAYLOAD), jnp.float32),
            jax.ShapeDtypeStruct((2, N_EDGES, PAYLOAD), jnp.float32),
        ],
        mesh=plsc.VectorSubcoreMesh(core_axis_name="c", subcore_axis_name="s"),
        scratch_types=[
            pltpu.VMEM((CHUNK, PAYLOAD), jnp.float32),
            pltpu.VMEM((CHUNK,), jnp.int32),
            pltpu.VMEM((SCATTERS, SCATTER_ROWS), jnp.int32),
            pltpu.VMEM((CHUNK,), jnp.float32),
            pltpu.VMEM_SHARED((N_NODES,), jnp.float32),
            pltpu.VMEM_SHARED((N_PAD, PAYLOAD), jnp.float32),
        ],
        compiler_params=pltpu.CompilerParams(
            needs_layout_passes=False, use_tc_tiling_on_sc=False),
    )
    return run(sums, sender, rcv3d, radial)


def kernel(edge_index, radial_feature, node_feats):
    e = radial_feature.shape[1]
    k, i, j = radial_feature.shape[2:]
    n = node_feats.shape[0]
    sender = edge_index[0].astype(jnp.int32)
    receiver = edge_index[1].astype(jnp.int32)
    radial = radial_feature.reshape(2, e, k * i * j)
    sums = _node_sums(node_feats)
    rcv3d = receiver.reshape(e // CHUNK, SCATTERS, SCATTER_ROWS)
    a, comb = _sc_main(sums, sender, rcv3d, radial)
    return (
        a[0, :n].reshape(n, k, i, j),
        a[1, :n].reshape(n, k, i, j),
        comb[0].reshape(e, k, i, j),
        comb[1].reshape(e, k, i, j),
    )
